# baseline (speedup 1.0000x reference)
import jax
import jax.numpy as jnp
from jax.experimental import pallas as pl
from jax.experimental.pallas import tpu as pltpu

_NB = 500
_BUFS = 2


def _add_kernel(x_ref, e_ref, o_ref):
    o_ref[...] = x_ref[...] + e_ref[...][None, :, None, :]


def kernel(x, emb_weight):
    B, N, T, D = x.shape
    return pl.pallas_call(
        _add_kernel,
        grid=(B, pl.cdiv(N, _NB)),
        in_specs=[
            pl.BlockSpec((1, _NB, T, D), lambda b, j: (b, j, 0, 0),
                         pipeline_mode=pl.Buffered(buffer_count=_BUFS)),
            pl.BlockSpec((_NB, D), lambda b, j: (j, 0)),
        ],
        out_specs=pl.BlockSpec((1, _NB, T, D), lambda b, j: (b, j, 0, 0),
                               pipeline_mode=pl.Buffered(buffer_count=_BUFS)),
        out_shape=jax.ShapeDtypeStruct((B, N, T, D), x.dtype),
        compiler_params=pltpu.CompilerParams(
            dimension_semantics=("parallel", "parallel"),
            skip_device_barrier=True,
            disable_bounds_checks=True,
        ),
    )(x, emb_weight)


# speedup vs baseline: 1.1117x; 1.1117x over previous
import jax
import jax.numpy as jnp
from jax.experimental import pallas as pl
from jax.experimental.pallas import tpu as pltpu

_K = 4


def _add_kernel(x_hbm, e_ref, o_hbm, in_buf, out_buf, in_sem, out_sem):
    B = x_hbm.shape[0]
    e = e_ref[...][:, None, :]

    def in_copy(i, slot):
        return pltpu.make_async_copy(x_hbm.at[i], in_buf.at[slot], in_sem.at[slot])

    def out_copy(i, slot):
        return pltpu.make_async_copy(out_buf.at[slot], o_hbm.at[i], out_sem.at[slot])

    for i in range(_K):
        in_copy(i, i).start()

    def body(i, _):
        si = jax.lax.rem(i, _K)
        in_copy(i, si).wait()

        @pl.when(i >= _K)
        def _():
            out_copy(i - _K, si).wait()

        out_buf[si] = in_buf[si] + e
        out_copy(i, si).start()

        @pl.when(i + _K < B)
        def _():
            in_copy(i + _K, si).start()

        return 0

    jax.lax.fori_loop(0, B, body, 0)
    for i in range(B - _K, B):
        out_copy(i, i % _K).wait()


def kernel(x, emb_weight):
    B, N, T, D = x.shape
    return pl.pallas_call(
        _add_kernel,
        in_specs=[
            pl.BlockSpec(memory_space=pltpu.HBM),
            pl.BlockSpec(memory_space=pltpu.VMEM),
        ],
        out_specs=pl.BlockSpec(memory_space=pltpu.HBM),
        out_shape=jax.ShapeDtypeStruct((B, N, T, D), x.dtype),
        scratch_shapes=[
            pltpu.VMEM((_K, N, T, D), x.dtype),
            pltpu.VMEM((_K, N, T, D), x.dtype),
            pltpu.SemaphoreType.DMA((_K,)),
            pltpu.SemaphoreType.DMA((_K,)),
        ],
        compiler_params=pltpu.CompilerParams(
            vmem_limit_bytes=100 * 1024 * 1024,
        ),
    )(x, emb_weight)
